# baseline (speedup 1.0000x reference)
import functools

import jax
import jax.numpy as jnp
from jax import lax
from jax.experimental import pallas as pl
from jax.experimental.pallas import tpu as pltpu
from jax.experimental.pallas import tpu_sc as plsc

N = 10000
E = 320000
R = 8
D = 128

NC = 2
NS = 16
NW = NC * NS

CH = 48
NCHUNK = 216
EPT = CH * NCHUNK
EPAD = NW * EPT
NP = 10112
NPT = NP // NS
TRASH = N


def _mesh():
    return plsc.VectorSubcoreMesh(
        core_axis_name="c", subcore_axis_name="s",
        num_cores=NC, num_subcores=NS)


def _count_body(pk, zref, cparts, *scr):
    pkb = scr[0:4]
    db = scr[4:8]
    rows = scr[8:12]
    semi = scr[12:16]
    sems = scr[16:20]
    acc = scr[20]
    sid = lax.axis_index("s")
    cid = lax.axis_index("c")
    lane = lax.iota(jnp.int32, 16)

    pltpu.sync_copy(zref.at[pl.ds(sid * NPT, NPT), :],
                    acc.at[pl.ds(sid * NPT, NPT), :])

    wid = cid * NS + sid
    base = pl.multiple_of(wid * EPT, CH)

    def _issue_idx(c, s):
        off = pl.multiple_of(base + c * CH, CH)
        pltpu.async_copy(pk.at[pl.ds(off, CH)], pkb[s], semi[s])

    def _wait_idx(c, s):
        off = pl.multiple_of(base + c * CH, CH)
        pltpu.make_async_copy(pk.at[pl.ds(off, CH)], pkb[s], semi[s]).wait()

    def _wait_scat(s):
        pltpu.make_async_copy(rows[s], acc.at[db[s]], sems[s]).wait()

    def _z(i, _):
        for t in range(4):
            for j in range(8):
                rows[t][i, pl.ds(j * 16, 16)] = jnp.zeros((16,), jnp.float32)
        return 0
    lax.fori_loop(0, CH, _z, 0)
    plsc.subcore_barrier()

    _issue_idx(0, 0)
    _issue_idx(1, 1)

    def _iter(k, _):
        for s in range(4):
            c = k * 4 + s
            s2 = (s + 2) % 4

            @pl.when(c >= 2)
            def _():
                _wait_scat(s2)

            @pl.when(c + 2 < NCHUNK)
            def _():
                _issue_idx(c + 2, s2)
            _wait_idx(c, s)

            def _mk(g, _):
                o = pl.multiple_of(g * 16, 16)
                v = pkb[s][pl.ds(o, 16)]
                tv = (lax.shift_right_logical(v, 14) - 1) % 9
                db[s][pl.ds(o, 16)] = v & 16383
                for l in range(16):
                    sv = jnp.full((16,), tv[l], jnp.int32)
                    rows[s][g * 16 + l, pl.ds(0, 16)] = jnp.where(
                        lane == sv, 1.0, 0.0).astype(jnp.float32)
                return 0
            lax.fori_loop(0, CH // 16, _mk, 0)
            pltpu.async_copy(rows[s], acc.at[db[s]], sems[s], add=True)
        return 0
    lax.fori_loop(0, NCHUNK // 4, _iter, 0)
    _wait_scat((NCHUNK - 2) % 4)
    _wait_scat((NCHUNK - 1) % 4)
    plsc.subcore_barrier()

    pltpu.sync_copy(acc.at[pl.ds(sid * NPT, NPT), :],
                    cparts.at[cid, pl.ds(sid * NPT, NPT), :])


@functools.lru_cache(maxsize=None)
def _get_count():
  return pl.kernel(
    _count_body,
    out_type=jax.ShapeDtypeStruct((NC, NP, D), jnp.float32),
    mesh=_mesh(),
    scratch_types=(
        [pltpu.VMEM((CH,), jnp.int32)] * 8
        + [pltpu.VMEM((CH, D), jnp.float32)] * 4
        + [pltpu.SemaphoreType.DMA] * 8
        + [pltpu.VMEM_SHARED((NP, D), jnp.float32)]
    ),
  )


def _wedge_body(tkeys, dstp, wtab, wvec, *scr):
    tb = scr[0:4]
    db = scr[4:8]
    w128 = scr[8:12]
    wout = scr[12:16]
    semi = scr[16:20]
    semg = scr[20:24]
    sems = scr[24:28]
    sid = lax.axis_index("s")
    cid = lax.axis_index("c")
    lane = lax.iota(jnp.int32, 16)

    wid = cid * NS + sid
    base = pl.multiple_of(wid * EPT, CH)

    def _off(c):
        return pl.multiple_of(base + c * CH, CH)

    def _issue_idx(c, s):
        pltpu.async_copy(tkeys.at[pl.ds(_off(c), CH)], tb[s], semi[s])
        pltpu.async_copy(dstp.at[pl.ds(_off(c), CH)], db[s], semi[s])

    def _wait_idx(c, s):
        pltpu.make_async_copy(tkeys.at[pl.ds(_off(c), CH)],
                              tb[s], semi[s]).wait()
        pltpu.make_async_copy(dstp.at[pl.ds(_off(c), CH)],
                              db[s], semi[s]).wait()

    def _wait_st(c, s):
        pltpu.make_async_copy(wout[s], wvec.at[pl.ds(_off(c), CH)],
                              sems[s]).wait()

    def _extract(s, g, _):
        tv = tb[s][pl.ds(pl.multiple_of(g * 16, 16), 16)]
        wv = jnp.zeros((16,), jnp.float32)
        for l in range(16):
            vv = w128[s][g * 16 + l, pl.ds(0, 16)]
            v = jnp.take(vv, jnp.full((16,), tv[l], jnp.int32))
            wv = jnp.where(lane == l, v, wv)
        wout[s][pl.ds(pl.multiple_of(g * 16, 16), 16)] = wv
        return 0

    _issue_idx(0, 0)
    _issue_idx(1, 1)
    _wait_idx(0, 0)
    pltpu.async_copy(wtab.at[db[0]], w128[0], semg[0])

    def _iter(k, _):
        for s in range(4):
            c = k * 4 + s
            s2 = (s + 2) % 4

            @pl.when(c >= 2)
            def _():
                _wait_st(c - 2, s2)

            @pl.when(c + 2 < NCHUNK)
            def _():
                _issue_idx(c + 2, s2)

            @pl.when(c + 1 < NCHUNK)
            def _():
                _wait_idx(c + 1, (s + 1) % 4)
                pltpu.async_copy(wtab.at[db[(s + 1) % 4]],
                                 w128[(s + 1) % 4], semg[(s + 1) % 4])
            pltpu.make_async_copy(wtab.at[db[s]], w128[s], semg[s]).wait()
            lax.fori_loop(0, CH // 16, functools.partial(_extract, s), 0)
            pltpu.async_copy(wout[s], wvec.at[pl.ds(_off(c), CH)], sems[s])
        return 0
    lax.fori_loop(0, NCHUNK // 4, _iter, 0)
    _wait_st(NCHUNK - 2, (NCHUNK - 2) % 4)
    _wait_st(NCHUNK - 1, (NCHUNK - 1) % 4)


@functools.lru_cache(maxsize=None)
def _get_wedge():
  return pl.kernel(
    _wedge_body,
    out_type=jax.ShapeDtypeStruct((EPAD,), jnp.float32),
    mesh=_mesh(),
    scratch_types=(
        [pltpu.VMEM((CH,), jnp.int32)] * 8
        + [pltpu.VMEM((CH, D), jnp.float32)] * 4
        + [pltpu.VMEM((CH,), jnp.float32)] * 4
        + [pltpu.SemaphoreType.DMA] * 12
    ),
  )


_NB = 6


def _layer_body(ytab, pk, wv, zref, part, *scr):
    pkb = scr[0:_NB]
    gb = scr[_NB:2 * _NB]
    db = scr[2 * _NB:3 * _NB]
    wb = scr[3 * _NB:4 * _NB]
    rows = scr[4 * _NB:5 * _NB]
    semi = scr[5 * _NB:6 * _NB]
    semg = scr[6 * _NB:7 * _NB]
    sems = scr[7 * _NB:8 * _NB]
    acc = scr[8 * _NB]
    sid = lax.axis_index("s")
    cid = lax.axis_index("c")

    pltpu.sync_copy(zref.at[pl.ds(sid * NPT, NPT), :],
                    acc.at[pl.ds(sid * NPT, NPT), :])
    plsc.subcore_barrier()

    wid = cid * NS + sid
    base = pl.multiple_of(wid * EPT, CH)

    def _issue_idx(c, s):
        off = pl.multiple_of(base + c * CH, CH)
        pltpu.async_copy(pk.at[pl.ds(off, CH)], pkb[s], semi[s])
        pltpu.async_copy(wv.at[pl.ds(off, CH)], wb[s], semi[s])

    def _wait_idx(c, s):
        off = pl.multiple_of(base + c * CH, CH)
        pltpu.make_async_copy(pk.at[pl.ds(off, CH)], pkb[s], semi[s]).wait()
        pltpu.make_async_copy(wv.at[pl.ds(off, CH)], wb[s], semi[s]).wait()
        def _unpack(g, _):
            o = pl.multiple_of(g * 16, 16)
            v = pkb[s][pl.ds(o, 16)]
            gb[s][pl.ds(o, 16)] = lax.shift_right_logical(v, 14)
            db[s][pl.ds(o, 16)] = v & 16383
            return 0
        lax.fori_loop(0, CH // 16, _unpack, 0)

    def _wait_scat(s):
        pltpu.make_async_copy(rows[s], acc.at[db[s]], sems[s]).wait()

    def _scale(s, g, _):
        wv16 = wb[s][pl.ds(pl.multiple_of(g * 16, 16), 16)]
        for l in range(16):
            sv = jnp.full((16,), wv16[l], jnp.float32)
            i = g * 16 + l
            rr = rows[s]
            for j in range(8):
                rr[i, pl.ds(j * 16, 16)] = rr[i, pl.ds(j * 16, 16)] * sv
        return 0

    _issue_idx(0, 0)
    _issue_idx(1, 1)
    _issue_idx(2, 2)
    _wait_idx(0, 0)
    pltpu.async_copy(ytab.at[gb[0]], rows[0], semg[0])
    _wait_idx(1, 1)
    pltpu.async_copy(ytab.at[gb[1]], rows[1], semg[1])

    def _iter(k, _):
        for s in range(_NB):
            c = k * _NB + s
            s3 = (s + 3) % _NB

            @pl.when(c >= 3)
            def _():
                _wait_scat(s3)

            @pl.when(c + 3 < NCHUNK)
            def _():
                _issue_idx(c + 3, s3)

            @pl.when(c + 2 < NCHUNK)
            def _():
                _wait_idx(c + 2, (s + 2) % _NB)
                pltpu.async_copy(ytab.at[gb[(s + 2) % _NB]],
                                 rows[(s + 2) % _NB], semg[(s + 2) % _NB])
            pltpu.make_async_copy(ytab.at[gb[s]], rows[s], semg[s]).wait()
            lax.fori_loop(0, CH // 16, functools.partial(_scale, s), 0)
            pltpu.async_copy(rows[s], acc.at[db[s]], sems[s], add=True)
        return 0
    lax.fori_loop(0, NCHUNK // _NB, _iter, 0)
    _wait_scat((NCHUNK - 3) % _NB)
    _wait_scat((NCHUNK - 2) % _NB)
    _wait_scat((NCHUNK - 1) % _NB)
    plsc.subcore_barrier()

    pltpu.sync_copy(acc.at[pl.ds(sid * NPT, NPT), :],
                    part.at[cid, pl.ds(sid * NPT, NPT), :])


@functools.lru_cache(maxsize=None)
def _get_sc_layer():
  return pl.kernel(
    _layer_body,
    out_type=jax.ShapeDtypeStruct((NC, NP, D), jnp.float32),
    mesh=_mesh(),
    scratch_types=(
        [pltpu.VMEM((CH,), jnp.int32)] * (3 * _NB)
        + [pltpu.VMEM((CH,), jnp.float32)] * _NB
        + [pltpu.VMEM((CH, D), jnp.float32)] * _NB
        + [pltpu.SemaphoreType.DMA] * (3 * _NB)
        + [pltpu.VMEM_SHARED((NP, D), jnp.float32)]
    ),
  )


_BM = 400
_GRID = N // _BM
_DW = (R + 1) * D


def _w_body(c0_ref, c1_ref, o_ref):
    c = c0_ref[0] + c1_ref[0]
    o_ref[...] = 1.0 / jnp.maximum(c, 1.0)


_w16 = pl.pallas_call(
    _w_body,
    grid=(NS,),
    in_specs=[
        pl.BlockSpec((1, NPT, D), lambda i: (0, i, 0)),
        pl.BlockSpec((1, NPT, D), lambda i: (1, i, 0)),
    ],
    out_specs=pl.BlockSpec((NPT, D), lambda i: (i, 0)),
    out_shape=jax.ShapeDtypeStruct((NP, D), jnp.float32),
)


def _mm0_body(x_ref, w_ref, o_ref):
    o_ref[...] = jnp.dot(x_ref[...], w_ref[...],
                         preferred_element_type=jnp.float32)


_mm0 = pl.pallas_call(
    _mm0_body,
    grid=(_GRID,),
    in_specs=[
        pl.BlockSpec((_BM, D), lambda i: (i, 0)),
        pl.BlockSpec((D, _DW), lambda i: (0, 0)),
    ],
    out_specs=pl.BlockSpec((_BM, _DW), lambda i: (i, 0)),
    out_shape=jax.ShapeDtypeStruct((N, _DW), jnp.float32),
)


def _mid_body(y_ref, p0_ref, p1_ref, b_ref, w_ref, o_ref):
    h = y_ref[...] + p0_ref[0] + p1_ref[0] + b_ref[...]
    h = jnp.maximum(h, 0.0)
    o_ref[...] = jnp.dot(h, w_ref[...], preferred_element_type=jnp.float32)


_mm_mid = pl.pallas_call(
    _mid_body,
    grid=(_GRID,),
    in_specs=[
        pl.BlockSpec((_BM, D), lambda i: (i, 0)),
        pl.BlockSpec((1, _BM, D), lambda i: (0, i, 0)),
        pl.BlockSpec((1, _BM, D), lambda i: (1, i, 0)),
        pl.BlockSpec((1, D), lambda i: (0, 0)),
        pl.BlockSpec((D, _DW), lambda i: (0, 0)),
    ],
    out_specs=pl.BlockSpec((_BM, _DW), lambda i: (i, 0)),
    out_shape=jax.ShapeDtypeStruct((N, _DW), jnp.float32),
)


def _fin_body(y_ref, p0_ref, p1_ref, b_ref, o_ref):
    o_ref[...] = y_ref[...] + p0_ref[0] + p1_ref[0] + b_ref[...]


_fin = pl.pallas_call(
    _fin_body,
    grid=(_GRID,),
    in_specs=[
        pl.BlockSpec((_BM, D), lambda i: (i, 0)),
        pl.BlockSpec((1, _BM, D), lambda i: (0, i, 0)),
        pl.BlockSpec((1, _BM, D), lambda i: (1, i, 0)),
        pl.BlockSpec((1, D), lambda i: (0, 0)),
    ],
    out_specs=pl.BlockSpec((_BM, D), lambda i: (i, 0)),
    out_shape=jax.ShapeDtypeStruct((N, D), jnp.float32),
)


def kernel(x, edge_index, edge_type, W1, root1, b1, W2, root2, b2,
           W3, root3, b3):
    src = edge_index[0]
    dst = edge_index[1]
    t = edge_type

    pad = EPAD - E
    srcp = jnp.concatenate([src, jnp.zeros((pad,), jnp.int32)])
    tp = jnp.concatenate([t, jnp.zeros((pad,), jnp.int32)])
    dstp = jnp.concatenate([dst, jnp.full((pad,), TRASH, jnp.int32)])

    gk = srcp * (R + 1) + 1 + tp
    pk = gk * 16384 + dstp

    zref = jnp.zeros((NP, D), jnp.float32)

    cparts = _get_count()(pk, zref)
    wtab = _w16(cparts, cparts)
    wvec = _get_wedge()(tp, dstp, wtab)

    def wcat(root, W):
        return jnp.concatenate(
            [root, W.transpose(1, 0, 2).reshape(D, R * D)], axis=1)

    _sc_layer = _get_sc_layer()

    y = _mm0(x, wcat(root1, W1))
    p = _sc_layer(y.reshape(N * (R + 1), D), pk, wvec, zref)
    y = _mm_mid(y, p, p, b1.reshape(1, D), wcat(root2, W2))
    p = _sc_layer(y.reshape(N * (R + 1), D), pk, wvec, zref)
    y = _mm_mid(y, p, p, b2.reshape(1, D), wcat(root3, W3))
    p = _sc_layer(y.reshape(N * (R + 1), D), pk, wvec, zref)
    return _fin(y, p, p, b3.reshape(1, D))


# speedup vs baseline: 1.5064x; 1.5064x over previous
import functools

import jax
import jax.numpy as jnp
from jax import lax
from jax.experimental import pallas as pl
from jax.experimental.pallas import tpu as pltpu
from jax.experimental.pallas import tpu_sc as plsc

N = 10000
E = 320000
R = 8
D = 128

NC = 2
NS = 16
NW = NC * NS

CH = 64
NCHUNK = 160
EPT = CH * NCHUNK
EPAD = NW * EPT
NP = 10112
NPT = NP // NS
TRASH = N


def _mesh():
    return plsc.VectorSubcoreMesh(
        core_axis_name="c", subcore_axis_name="s",
        num_cores=NC, num_subcores=NS)


def _count_body(pk, zref, cparts, *scr):
    pkb = scr[0:4]
    db = scr[4:8]
    rows = scr[8:12]
    semi = scr[12:16]
    sems = scr[16:20]
    acc = scr[20]
    sid = lax.axis_index("s")
    cid = lax.axis_index("c")
    lane = lax.iota(jnp.int32, 16)

    pltpu.sync_copy(zref.at[pl.ds(sid * NPT, NPT), :],
                    acc.at[pl.ds(sid * NPT, NPT), :])

    wid = cid * NS + sid
    base = pl.multiple_of(wid * EPT, CH)

    def _issue_idx(c, s):
        off = pl.multiple_of(base + c * CH, CH)
        pltpu.async_copy(pk.at[pl.ds(off, CH)], pkb[s], semi[s])

    def _wait_idx(c, s):
        off = pl.multiple_of(base + c * CH, CH)
        pltpu.make_async_copy(pk.at[pl.ds(off, CH)], pkb[s], semi[s]).wait()

    def _wait_scat(s):
        pltpu.make_async_copy(rows[s], acc.at[db[s]], sems[s]).wait()

    def _z(i, _):
        for t in range(4):
            for j in range(8):
                rows[t][i, pl.ds(j * 16, 16)] = jnp.zeros((16,), jnp.float32)
        return 0
    lax.fori_loop(0, CH, _z, 0)
    plsc.subcore_barrier()

    _issue_idx(0, 0)
    _issue_idx(1, 1)

    def _iter(k, _):
        for s in range(4):
            c = k * 4 + s
            s2 = (s + 2) % 4

            @pl.when(c >= 2)
            def _():
                _wait_scat(s2)

            @pl.when(c + 2 < NCHUNK)
            def _():
                _issue_idx(c + 2, s2)
            _wait_idx(c, s)

            def _mk(g, _):
                o = pl.multiple_of(g * 16, 16)
                v = pkb[s][pl.ds(o, 16)]
                tv = (lax.shift_right_logical(v, 14) - 1) % 9
                db[s][pl.ds(o, 16)] = v & 16383
                for l in range(16):
                    sv = jnp.full((16,), tv[l], jnp.int32)
                    rows[s][g * 16 + l, pl.ds(0, 16)] = jnp.where(
                        lane == sv, 1.0, 0.0).astype(jnp.float32)
                return 0
            lax.fori_loop(0, CH // 16, _mk, 0)
            pltpu.async_copy(rows[s], acc.at[db[s]], sems[s], add=True)
        return 0
    lax.fori_loop(0, NCHUNK // 4, _iter, 0)
    _wait_scat((NCHUNK - 2) % 4)
    _wait_scat((NCHUNK - 1) % 4)
    plsc.subcore_barrier()

    pltpu.sync_copy(acc.at[pl.ds(sid * NPT, NPT), :],
                    cparts.at[cid, pl.ds(sid * NPT, NPT), :])


@functools.lru_cache(maxsize=None)
def _get_count():
  return pl.kernel(
    _count_body,
    out_type=jax.ShapeDtypeStruct((NC, NP, D), jnp.float32),
    mesh=_mesh(),
    scratch_types=(
        [pltpu.VMEM((CH,), jnp.int32)] * 8
        + [pltpu.VMEM((CH, D), jnp.float32)] * 4
        + [pltpu.SemaphoreType.DMA] * 8
        + [pltpu.VMEM_SHARED((NP, D), jnp.float32)]
    ),
  )


def _wedge_body(tkeys, dstp, wtab, wvec, *scr):
    tb = scr[0:4]
    db = scr[4:8]
    w128 = scr[8:12]
    wout = scr[12:16]
    semi = scr[16:20]
    semg = scr[20:24]
    sems = scr[24:28]
    sid = lax.axis_index("s")
    cid = lax.axis_index("c")
    lane = lax.iota(jnp.int32, 16)

    wid = cid * NS + sid
    base = pl.multiple_of(wid * EPT, CH)

    def _off(c):
        return pl.multiple_of(base + c * CH, CH)

    def _issue_idx(c, s):
        pltpu.async_copy(tkeys.at[pl.ds(_off(c), CH)], tb[s], semi[s])
        pltpu.async_copy(dstp.at[pl.ds(_off(c), CH)], db[s], semi[s])

    def _wait_idx(c, s):
        pltpu.make_async_copy(tkeys.at[pl.ds(_off(c), CH)],
                              tb[s], semi[s]).wait()
        pltpu.make_async_copy(dstp.at[pl.ds(_off(c), CH)],
                              db[s], semi[s]).wait()

    def _wait_st(c, s):
        pltpu.make_async_copy(wout[s], wvec.at[pl.ds(_off(c), CH)],
                              sems[s]).wait()

    def _extract(s, g, _):
        tv = tb[s][pl.ds(pl.multiple_of(g * 16, 16), 16)]
        wv = jnp.zeros((16,), jnp.float32)
        for l in range(16):
            vv = w128[s][g * 16 + l, pl.ds(0, 16)]
            v = jnp.take(vv, jnp.full((16,), tv[l], jnp.int32))
            wv = jnp.where(lane == l, v, wv)
        wout[s][pl.ds(pl.multiple_of(g * 16, 16), 16)] = wv
        return 0

    _issue_idx(0, 0)
    _issue_idx(1, 1)
    _wait_idx(0, 0)
    pltpu.async_copy(wtab.at[db[0]], w128[0], semg[0])

    def _iter(k, _):
        for s in range(4):
            c = k * 4 + s
            s2 = (s + 2) % 4

            @pl.when(c >= 2)
            def _():
                _wait_st(c - 2, s2)

            @pl.when(c + 2 < NCHUNK)
            def _():
                _issue_idx(c + 2, s2)

            @pl.when(c + 1 < NCHUNK)
            def _():
                _wait_idx(c + 1, (s + 1) % 4)
                pltpu.async_copy(wtab.at[db[(s + 1) % 4]],
                                 w128[(s + 1) % 4], semg[(s + 1) % 4])
            pltpu.make_async_copy(wtab.at[db[s]], w128[s], semg[s]).wait()
            lax.fori_loop(0, CH // 16, functools.partial(_extract, s), 0)
            pltpu.async_copy(wout[s], wvec.at[pl.ds(_off(c), CH)], sems[s])
        return 0
    lax.fori_loop(0, NCHUNK // 4, _iter, 0)
    _wait_st(NCHUNK - 2, (NCHUNK - 2) % 4)
    _wait_st(NCHUNK - 1, (NCHUNK - 1) % 4)


@functools.lru_cache(maxsize=None)
def _get_wedge():
  return pl.kernel(
    _wedge_body,
    out_type=jax.ShapeDtypeStruct((EPAD,), jnp.float32),
    mesh=_mesh(),
    scratch_types=(
        [pltpu.VMEM((CH,), jnp.int32)] * 8
        + [pltpu.VMEM((CH, D), jnp.float32)] * 4
        + [pltpu.VMEM((CH,), jnp.float32)] * 4
        + [pltpu.SemaphoreType.DMA] * 12
    ),
  )


_NB = 4


def _layer_body(ytab, edata, zref, part, *scr):
    eb = scr[0:_NB]
    gb = scr[_NB:2 * _NB]
    db = scr[2 * _NB:3 * _NB]
    rows = scr[3 * _NB:4 * _NB]
    semi = scr[4 * _NB:5 * _NB]
    semg = scr[5 * _NB:6 * _NB]
    sems = scr[6 * _NB:7 * _NB]
    acc = scr[7 * _NB]
    sid = lax.axis_index("s")
    cid = lax.axis_index("c")

    pltpu.sync_copy(zref.at[pl.ds(sid * NPT, NPT), :],
                    acc.at[pl.ds(sid * NPT, NPT), :])
    plsc.subcore_barrier()

    wid = cid * NS + sid
    base = pl.multiple_of(wid * EPT, CH)

    def _issue_idx(c, s):
        off = pl.multiple_of((base + c * CH) * 2, 2 * CH)
        pltpu.async_copy(edata.at[pl.ds(off, 2 * CH)], eb[s], semi[s])

    def _wait_idx(c, s):
        off = pl.multiple_of((base + c * CH) * 2, 2 * CH)
        pltpu.make_async_copy(edata.at[pl.ds(off, 2 * CH)],
                              eb[s], semi[s]).wait()
        def _unpack(g, _):
            o = pl.multiple_of(g * 16, 16)
            v = eb[s][pl.ds(o, 16)]
            gb[s][pl.ds(o, 16)] = lax.shift_right_logical(v, 14)
            db[s][pl.ds(o, 16)] = v & 16383
            return 0
        lax.fori_loop(0, CH // 16, _unpack, 0)

    def _wait_scat(s):
        pltpu.make_async_copy(rows[s], acc.at[db[s]], sems[s]).wait()

    def _scale(s, g, _):
        wv16 = lax.bitcast_convert_type(
            eb[s][pl.ds(pl.multiple_of(CH + g * 16, 16), 16)], jnp.float32)
        for l in range(16):
            sv = jnp.full((16,), wv16[l], jnp.float32)
            i = g * 16 + l
            rr = rows[s]
            for j in range(8):
                rr[i, pl.ds(j * 16, 16)] = rr[i, pl.ds(j * 16, 16)] * sv
        return 0

    _issue_idx(0, 0)
    _issue_idx(1, 1)
    _wait_idx(0, 0)
    pltpu.async_copy(ytab.at[gb[0]], rows[0], semg[0])

    def _iter(k, _):
        for s in range(_NB):
            c = k * _NB + s
            s2 = (s + 2) % _NB

            @pl.when(c >= 2)
            def _():
                _wait_scat(s2)

            @pl.when(c + 2 < NCHUNK)
            def _():
                _issue_idx(c + 2, s2)

            @pl.when(c + 1 < NCHUNK)
            def _():
                _wait_idx(c + 1, (s + 1) % _NB)
                pltpu.async_copy(ytab.at[gb[(s + 1) % _NB]],
                                 rows[(s + 1) % _NB], semg[(s + 1) % _NB])
            pltpu.make_async_copy(ytab.at[gb[s]], rows[s], semg[s]).wait()
            lax.fori_loop(0, CH // 16, functools.partial(_scale, s), 0)
            pltpu.async_copy(rows[s], acc.at[db[s]], sems[s], add=True)
        return 0
    lax.fori_loop(0, NCHUNK // _NB, _iter, 0)
    _wait_scat((NCHUNK - 2) % _NB)
    _wait_scat((NCHUNK - 1) % _NB)
    plsc.subcore_barrier()

    pltpu.sync_copy(acc.at[pl.ds(sid * NPT, NPT), :],
                    part.at[cid, pl.ds(sid * NPT, NPT), :])


@functools.lru_cache(maxsize=None)
def _get_sc_layer():
  return pl.kernel(
    _layer_body,
    out_type=jax.ShapeDtypeStruct((NC, NP, D), jnp.float32),
    mesh=_mesh(),
    scratch_types=(
        [pltpu.VMEM((2 * CH,), jnp.int32)] * _NB
        + [pltpu.VMEM((CH,), jnp.int32)] * (2 * _NB)
        + [pltpu.VMEM((CH, D), jnp.float32)] * _NB
        + [pltpu.SemaphoreType.DMA] * (3 * _NB)
        + [pltpu.VMEM_SHARED((NP, D), jnp.float32)]
    ),
  )


_BM = 400
_GRID = N // _BM
_DW = (R + 1) * D


def _w_body(c0_ref, c1_ref, o_ref):
    c = c0_ref[0] + c1_ref[0]
    o_ref[...] = 1.0 / jnp.maximum(c, 1.0)


_w16 = pl.pallas_call(
    _w_body,
    grid=(NS,),
    in_specs=[
        pl.BlockSpec((1, NPT, D), lambda i: (0, i, 0)),
        pl.BlockSpec((1, NPT, D), lambda i: (1, i, 0)),
    ],
    out_specs=pl.BlockSpec((NPT, D), lambda i: (i, 0)),
    out_shape=jax.ShapeDtypeStruct((NP, D), jnp.float32),
)


def _mm0_body(x_ref, w_ref, o_ref):
    o_ref[...] = jnp.dot(x_ref[...], w_ref[...],
                         preferred_element_type=jnp.float32)


_mm0 = pl.pallas_call(
    _mm0_body,
    grid=(_GRID,),
    in_specs=[
        pl.BlockSpec((_BM, D), lambda i: (i, 0)),
        pl.BlockSpec((D, _DW), lambda i: (0, 0)),
    ],
    out_specs=pl.BlockSpec((_BM, _DW), lambda i: (i, 0)),
    out_shape=jax.ShapeDtypeStruct((N, _DW), jnp.float32),
)


def _mid_body(y_ref, p0_ref, p1_ref, b_ref, w_ref, o_ref):
    h = y_ref[...] + p0_ref[0] + p1_ref[0] + b_ref[...]
    h = jnp.maximum(h, 0.0)
    o_ref[...] = jnp.dot(h, w_ref[...], preferred_element_type=jnp.float32)


_mm_mid = pl.pallas_call(
    _mid_body,
    grid=(_GRID,),
    in_specs=[
        pl.BlockSpec((_BM, D), lambda i: (i, 0)),
        pl.BlockSpec((1, _BM, D), lambda i: (0, i, 0)),
        pl.BlockSpec((1, _BM, D), lambda i: (1, i, 0)),
        pl.BlockSpec((1, D), lambda i: (0, 0)),
        pl.BlockSpec((D, _DW), lambda i: (0, 0)),
    ],
    out_specs=pl.BlockSpec((_BM, _DW), lambda i: (i, 0)),
    out_shape=jax.ShapeDtypeStruct((N, _DW), jnp.float32),
)


def _fin_body(y_ref, p0_ref, p1_ref, b_ref, o_ref):
    o_ref[...] = y_ref[...] + p0_ref[0] + p1_ref[0] + b_ref[...]


_fin = pl.pallas_call(
    _fin_body,
    grid=(_GRID,),
    in_specs=[
        pl.BlockSpec((_BM, D), lambda i: (i, 0)),
        pl.BlockSpec((1, _BM, D), lambda i: (0, i, 0)),
        pl.BlockSpec((1, _BM, D), lambda i: (1, i, 0)),
        pl.BlockSpec((1, D), lambda i: (0, 0)),
    ],
    out_specs=pl.BlockSpec((_BM, D), lambda i: (i, 0)),
    out_shape=jax.ShapeDtypeStruct((N, D), jnp.float32),
)


def kernel(x, edge_index, edge_type, W1, root1, b1, W2, root2, b2,
           W3, root3, b3):
    src = edge_index[0]
    dst = edge_index[1]
    t = edge_type

    pad = EPAD - E
    srcp = jnp.concatenate([src, jnp.zeros((pad,), jnp.int32)])
    tp = jnp.concatenate([t, jnp.zeros((pad,), jnp.int32)])
    dstp = jnp.concatenate([dst, jnp.full((pad,), TRASH, jnp.int32)])

    gk = srcp * (R + 1) + 1 + tp
    pk = gk * 16384 + dstp

    zref = jnp.zeros((NP, D), jnp.float32)

    cparts = _get_count()(pk, zref)
    wtab = _w16(cparts, cparts)
    wvec = _get_wedge()(tp, dstp, wtab)
    ed2 = jnp.stack([pk.reshape(-1, CH),
                     wvec.view(jnp.int32).reshape(-1, CH)], axis=1)
    edata = ed2.reshape(-1)

    def wcat(root, W):
        return jnp.concatenate(
            [root, W.transpose(1, 0, 2).reshape(D, R * D)], axis=1)

    _sc_layer = _get_sc_layer()

    y = _mm0(x, wcat(root1, W1))
    p = _sc_layer(y.reshape(N * (R + 1), D), edata, zref)
    y = _mm_mid(y, p, p, b1.reshape(1, D), wcat(root2, W2))
    p = _sc_layer(y.reshape(N * (R + 1), D), edata, zref)
    y = _mm_mid(y, p, p, b2.reshape(1, D), wcat(root3, W3))
    p = _sc_layer(y.reshape(N * (R + 1), D), edata, zref)
    return _fin(y, p, p, b3.reshape(1, D))
